# Initial kernel scaffold; baseline (speedup 1.0000x reference)
import jax
import jax.numpy as jnp
from jax.experimental import pallas as pl
from jax.experimental.pallas import tpu as pltpu

_NE = 512
_D = 64
_B = 64
_N = 1024
_NTOK = _B * _N
_R = 2048
_NB = _NTOK // _R


def _vq_block(z_ref, w_ref, q_ref, idx_ref, perp_ref, uniq_ref, counts_ref):
    i = pl.program_id(0)
    nb = pl.num_programs(0)
    zb = z_ref[...]
    w = w_ref[...]

    szz = jnp.sum(zb * zb, axis=1, keepdims=True)
    sww = jax.lax.dot_general(jnp.ones((1, _D), jnp.float32), w * w,
                              (((1,), (1,)), ((), ())))
    scores = jax.lax.dot_general(zb, w, (((1,), (1,)), ((), ())))
    dist = (szz + sww) - 2.0 * scores

    dmin = jnp.min(dist, axis=1, keepdims=True)
    lane = jax.lax.broadcasted_iota(jnp.int32, dist.shape, 1)
    idx = jnp.min(jnp.where(dist == dmin, lane, _NE), axis=1, keepdims=True)

    onehot = (lane == idx)
    qb = jax.lax.dot_general(onehot.astype(jnp.bfloat16), w.astype(jnp.bfloat16),
                             (((1,), (0,)), ((), ())),
                             preferred_element_type=jnp.float32)
    q_ref[...] = zb + (qb - zb)
    idx_ref[...] = idx

    @pl.when(i == 0)
    def _init():
        counts_ref[...] = jnp.zeros_like(counts_ref)

    counts_ref[...] += jnp.sum(onehot.astype(jnp.float32), axis=0, keepdims=True)

    @pl.when(i == nb - 1)
    def _fin():
        counts = counts_ref[...]
        avg = counts * (1.0 / _NTOK)
        ent = jnp.sum(avg * jnp.log(avg + 1e-10))
        perp_ref[0, 0] = jnp.exp(-ent)
        uniq_ref[0, 0] = jnp.sum((counts > 0.0).astype(jnp.int32))


def kernel(z, W):
    original_dtype = z.dtype
    zf = z.astype(jnp.float32).reshape(_NTOK, _D)
    q, idx, perp, uniq = pl.pallas_call(
        _vq_block,
        grid=(_NB,),
        in_specs=[
            pl.BlockSpec((_R, _D), lambda i: (i, 0)),
            pl.BlockSpec((_NE, _D), lambda i: (0, 0)),
        ],
        out_specs=[
            pl.BlockSpec((_R, _D), lambda i: (i, 0)),
            pl.BlockSpec((_R, 1), lambda i: (i, 0)),
            pl.BlockSpec((1, 1), lambda i: (0, 0)),
            pl.BlockSpec((1, 1), lambda i: (0, 0)),
        ],
        out_shape=[
            jax.ShapeDtypeStruct((_NTOK, _D), jnp.float32),
            jax.ShapeDtypeStruct((_NTOK, 1), jnp.int32),
            jax.ShapeDtypeStruct((1, 1), jnp.float32),
            jax.ShapeDtypeStruct((1, 1), jnp.int32),
        ],
        scratch_shapes=[pltpu.VMEM((1, _NE), jnp.float32)],
        compiler_params=pltpu.CompilerParams(
            dimension_semantics=("arbitrary",)),
    )(zf, W)
    quantized = q.reshape(_B, _N, _D).astype(original_dtype)
    indices = idx.reshape(_B, _N)
    vq_loss = jnp.zeros((), jnp.float32)
    return (quantized, indices, vq_loss, perp.reshape(()), uniq.reshape(()))


# speedup vs baseline: 2.1379x; 2.1379x over previous
import jax
import jax.numpy as jnp
from jax.experimental import pallas as pl
from jax.experimental.pallas import tpu as pltpu

_NE = 512
_D = 64
_B = 64
_N = 1024
_NTOK = _B * _N
_R = 2048
_NB = _NTOK // _R


def _vq_block(z_ref, w_ref, q_ref, idx_ref, perp_ref, uniq_ref, counts_ref):
    i = pl.program_id(0)
    nb = pl.num_programs(0)
    zb = z_ref[...]
    w = w_ref[...]

    szz = jnp.sum(zb * zb, axis=1, keepdims=True)
    sww = jax.lax.dot_general(jnp.ones((1, _D), jnp.float32), w * w,
                              (((1,), (1,)), ((), ())))
    scores = jax.lax.dot_general(zb, w, (((1,), (1,)), ((), ())))
    dist = (szz + sww) - 2.0 * scores

    dmin = jnp.min(dist, axis=1, keepdims=True)
    lane = jax.lax.broadcasted_iota(jnp.int32, dist.shape, 1)
    idx = jnp.min(jnp.where(dist == dmin, lane, _NE), axis=1, keepdims=True)

    onehot = (lane == idx)
    qb = jax.lax.dot_general(onehot.astype(jnp.bfloat16), w.astype(jnp.bfloat16),
                             (((1,), (0,)), ((), ())),
                             preferred_element_type=jnp.float32)
    q_ref[...] = zb + (qb - zb)
    idx_ref[...] = idx

    @pl.when(i == 0)
    def _init():
        counts_ref[...] = jnp.zeros_like(counts_ref)

    counts_ref[...] += jnp.sum(onehot.astype(jnp.float32), axis=0, keepdims=True)

    @pl.when(i == nb - 1)
    def _fin():
        counts = counts_ref[...]
        avg = counts * (1.0 / _NTOK)
        ent = jnp.sum(avg * jnp.log(avg + 1e-10), axis=(0, 1), keepdims=True)
        perp_ref[...] = jnp.exp(-ent)
        uniq_ref[...] = jnp.sum((counts > 0.0).astype(jnp.int32),
                                axis=(0, 1), keepdims=True)


def kernel(z, W):
    original_dtype = z.dtype
    zf = z.astype(jnp.float32).reshape(_NTOK, _D)
    q, idx, perp, uniq = pl.pallas_call(
        _vq_block,
        grid=(_NB,),
        in_specs=[
            pl.BlockSpec((_R, _D), lambda i: (i, 0)),
            pl.BlockSpec((_NE, _D), lambda i: (0, 0)),
        ],
        out_specs=[
            pl.BlockSpec((_R, _D), lambda i: (i, 0)),
            pl.BlockSpec((_R, 1), lambda i: (i, 0)),
            pl.BlockSpec((1, 1), lambda i: (0, 0)),
            pl.BlockSpec((1, 1), lambda i: (0, 0)),
        ],
        out_shape=[
            jax.ShapeDtypeStruct((_NTOK, _D), jnp.float32),
            jax.ShapeDtypeStruct((_NTOK, 1), jnp.int32),
            jax.ShapeDtypeStruct((1, 1), jnp.float32),
            jax.ShapeDtypeStruct((1, 1), jnp.int32),
        ],
        scratch_shapes=[pltpu.VMEM((1, _NE), jnp.float32)],
        compiler_params=pltpu.CompilerParams(
            dimension_semantics=("arbitrary",)),
    )(zf, W)
    quantized = q.reshape(_B, _N, _D).astype(original_dtype)
    indices = idx.reshape(_B, _N)
    vq_loss = jnp.zeros((), jnp.float32)
    return (quantized, indices, vq_loss, perp.reshape(()), uniq.reshape(()))


# speedup vs baseline: 2.3235x; 1.0868x over previous
import jax
import jax.numpy as jnp
from jax.experimental import pallas as pl
from jax.experimental.pallas import tpu as pltpu

_NE = 512
_D = 64
_B = 64
_N = 1024
_NTOK = _B * _N
_R = 2048
_NB = _NTOK // _R


def _vq_block(z_ref, w_ref, q_ref, idx_ref, perp_ref, uniq_ref,
              sww_ref, wm2_ref, wbf_ref, counts_ref):
    i = pl.program_id(0)
    nb = pl.num_programs(0)

    @pl.when(i == 0)
    def _prep():
        w = w_ref[...]
        sww_ref[...] = jax.lax.dot_general(
            jnp.ones((1, _D), jnp.float32), w * w, (((1,), (1,)), ((), ())))
        wm2_ref[...] = w * (-2.0)
        wbf_ref[...] = w.astype(jnp.bfloat16)
        counts_ref[...] = jnp.zeros_like(counts_ref)

    zb = z_ref[...]

    szz = jnp.sum(zb * zb, axis=1, keepdims=True)
    scores2 = jax.lax.dot_general(zb, wm2_ref[...],
                                  (((1,), (1,)), ((), ())))
    dist = (szz + sww_ref[...]) + scores2

    dmin = jnp.min(dist, axis=1, keepdims=True)
    lane = jax.lax.broadcasted_iota(jnp.int32, dist.shape, 1)
    idx = jnp.min(jnp.where(dist == dmin, lane, _NE),
                  axis=1, keepdims=True)

    onehot = (lane == idx).astype(jnp.bfloat16)
    qb = jax.lax.dot_general(onehot, wbf_ref[...], (((1,), (0,)), ((), ())),
                             preferred_element_type=jnp.float32)
    q_ref[...] = zb + (qb - zb)
    idx_ref[...] = idx.reshape(1, 1, _R)

    counts_ref[...] += jax.lax.dot_general(
        jnp.ones((1, _R), jnp.bfloat16), onehot, (((1,), (0,)), ((), ())),
        preferred_element_type=jnp.float32)

    @pl.when(i == nb - 1)
    def _fin():
        counts = counts_ref[...]
        avg = counts * (1.0 / _NTOK)
        ent = jnp.sum(avg * jnp.log(avg + 1e-10), axis=(0, 1), keepdims=True)
        perp_ref[...] = jnp.exp(-ent)
        uniq_ref[...] = jnp.sum((counts > 0.0).astype(jnp.int32),
                                axis=(0, 1), keepdims=True)


def kernel(z, W):
    original_dtype = z.dtype
    zf = z.astype(jnp.float32).reshape(_NTOK, _D)
    q, idx, perp, uniq = pl.pallas_call(
        _vq_block,
        grid=(_NB,),
        in_specs=[
            pl.BlockSpec((_R, _D), lambda i: (i, 0)),
            pl.BlockSpec((_NE, _D), lambda i: (0, 0)),
        ],
        out_specs=[
            pl.BlockSpec((_R, _D), lambda i: (i, 0)),
            pl.BlockSpec((1, 1, _R), lambda i: (i, 0, 0)),
            pl.BlockSpec((1, 1), lambda i: (0, 0)),
            pl.BlockSpec((1, 1), lambda i: (0, 0)),
        ],
        out_shape=[
            jax.ShapeDtypeStruct((_NTOK, _D), jnp.float32),
            jax.ShapeDtypeStruct((_NB, 1, _R), jnp.int32),
            jax.ShapeDtypeStruct((1, 1), jnp.float32),
            jax.ShapeDtypeStruct((1, 1), jnp.int32),
        ],
        scratch_shapes=[
            pltpu.VMEM((1, _NE), jnp.float32),
            pltpu.VMEM((_NE, _D), jnp.float32),
            pltpu.VMEM((_NE, _D), jnp.bfloat16),
            pltpu.VMEM((1, _NE), jnp.float32),
        ],
        compiler_params=pltpu.CompilerParams(
            dimension_semantics=("arbitrary",)),
    )(zf, W)
    quantized = q.reshape(_B, _N, _D).astype(original_dtype)
    indices = idx.reshape(_B, _N)
    vq_loss = jnp.zeros((), jnp.float32)
    return (quantized, indices, vq_loss, perp.reshape(()), uniq.reshape(()))
